# baseline (device time: 162860 ns/iter reference)
import jax
import jax.numpy as jnp
from jax import lax
from jax.experimental import pallas as pl
from jax.experimental.pallas import tpu as pltpu

N_DEV = 4
N_LOCAL_EXPERTS = 4


def kernel(x, router_W, route_idx, expert_W):
    n_tok, d_model = x.shape
    n_experts = router_W.shape[1]
    d_out = expert_W.shape[2]

    def body(x_ref, rw_ref, idx_ref, ew_ref, out_ref,
             comm_ref, send_sems, recv_sems):
        my_pos = lax.axis_index("i")
        left = lax.rem(my_pos - 1 + N_DEV, N_DEV)
        right = lax.rem(my_pos + 1, N_DEV)

        barrier_sem = pltpu.get_barrier_semaphore()
        for nbr in (left, right):
            pl.semaphore_signal(
                barrier_sem, inc=1,
                device_id=(nbr,), device_id_type=pl.DeviceIdType.MESH,
            )
        pl.semaphore_wait(barrier_sem, 2)

        xf = x_ref[:, :]
        scores = lax.dot_general(
            xf, rw_ref[:, :], (((1,), (0,)), ((), ())),
            preferred_element_type=jnp.float32,
        )
        s_max = jnp.max(scores, axis=1, keepdims=True)
        probs = jnp.exp(scores - s_max)
        probs = probs / jnp.sum(probs, axis=1, keepdims=True)

        e0 = idx_ref[:, 0:1]
        e1 = idx_ref[:, 1:2]
        expert_iota = lax.broadcasted_iota(jnp.int32, (n_tok, n_experts), 1)
        g0 = jnp.sum(
            jnp.where(e0 == expert_iota, probs, 0.0), axis=1, keepdims=True
        )
        g1 = jnp.sum(
            jnp.where(e1 == expert_iota, probs, 0.0), axis=1, keepdims=True
        )
        gs = g0 + g1
        g0n = g0 / gs
        g1n = g1 / gs

        partial = jnp.zeros((n_tok, d_out), dtype=jnp.float32)
        for j in range(N_LOCAL_EXPERTS):
            gid = my_pos * N_LOCAL_EXPERTS + j
            w_j = (
                g0n * (e0 == gid).astype(jnp.float32)
                + g1n * (e1 == gid).astype(jnp.float32)
            )
            xw = (xf * w_j).astype(jnp.bfloat16)
            W_j = ew_ref[j, :, :].astype(jnp.bfloat16)
            partial = partial + lax.dot_general(
                xw, W_j, (((1,), (0,)), ((), ())),
                preferred_element_type=jnp.float32,
            )

        out_ref[:, :] = partial
        comm_ref[0, :, :] = partial

        for h in range(N_DEV - 1):
            rdma = pltpu.make_async_remote_copy(
                src_ref=comm_ref.at[h],
                dst_ref=comm_ref.at[h + 1],
                send_sem=send_sems.at[h],
                recv_sem=recv_sems.at[h],
                device_id=(right,),
                device_id_type=pl.DeviceIdType.MESH,
            )
            rdma.start()
            rdma.wait()
            out_ref[:, :] += comm_ref[h + 1, :, :]

    return pl.pallas_call(
        body,
        out_shape=jax.ShapeDtypeStruct((n_tok, d_out), jnp.float32),
        in_specs=[
            pl.BlockSpec(memory_space=pltpu.VMEM),
            pl.BlockSpec(memory_space=pltpu.VMEM),
            pl.BlockSpec(memory_space=pltpu.VMEM),
            pl.BlockSpec(memory_space=pltpu.VMEM),
        ],
        out_specs=pl.BlockSpec(memory_space=pltpu.VMEM),
        scratch_shapes=[
            pltpu.VMEM((N_DEV, n_tok, d_out), jnp.float32),
            pltpu.SemaphoreType.DMA((N_DEV - 1,)),
            pltpu.SemaphoreType.DMA((N_DEV - 1,)),
        ],
        compiler_params=pltpu.CompilerParams(collective_id=0),
    )(x, router_W, route_idx, expert_W)


# device time: 96900 ns/iter; 1.6807x vs baseline; 1.6807x over previous
import jax
import jax.numpy as jnp
from jax import lax
from jax.experimental import pallas as pl
from jax.experimental.pallas import tpu as pltpu

N_DEV = 4
N_LOCAL_EXPERTS = 4


def kernel(x, router_W, route_idx, expert_W):
    n_tok, d_model = x.shape
    n_experts = router_W.shape[1]
    d_out = expert_W.shape[2]
    chunk = d_out // N_DEV

    def body(x_ref, rw_ref, idx_ref, ew_ref, out_ref,
             send_buf, recv_buf, ag_buf,
             rs_send, rs_recv, ag_send, ag_recv):
        my = lax.axis_index("i")
        left = lax.rem(my + N_DEV - 1, N_DEV)
        right = lax.rem(my + 1, N_DEV)

        barrier_sem = pltpu.get_barrier_semaphore()
        for nbr in (left, right):
            pl.semaphore_signal(
                barrier_sem, inc=1,
                device_id=(nbr,), device_id_type=pl.DeviceIdType.MESH,
            )
        pl.semaphore_wait(barrier_sem, 2)

        xf = x_ref[:, :]
        scores = lax.dot_general(
            xf, rw_ref[:, :], (((1,), (0,)), ((), ())),
            preferred_element_type=jnp.float32,
        )
        s_max = jnp.max(scores, axis=1, keepdims=True)
        probs = jnp.exp(scores - s_max)
        probs = probs / jnp.sum(probs, axis=1, keepdims=True)

        e0 = idx_ref[:, 0:1]
        e1 = idx_ref[:, 1:2]
        expert_iota = lax.broadcasted_iota(jnp.int32, (n_tok, n_experts), 1)
        g0 = jnp.sum(
            jnp.where(e0 == expert_iota, probs, 0.0), axis=1, keepdims=True
        )
        g1 = jnp.sum(
            jnp.where(e1 == expert_iota, probs, 0.0), axis=1, keepdims=True
        )
        gs = g0 + g1
        g0n = g0 / gs
        g1n = g1 / gs

        xw = []
        for j in range(N_LOCAL_EXPERTS):
            gid = my * N_LOCAL_EXPERTS + j
            w_j = (
                g0n * (e0 == gid).astype(jnp.float32)
                + g1n * (e1 == gid).astype(jnp.float32)
            )
            xw.append((xf * w_j).astype(jnp.bfloat16))

        def pchunk(c):
            acc = jnp.zeros((n_tok, chunk), dtype=jnp.float32)
            for j in range(N_LOCAL_EXPERTS):
                W_j = ew_ref[j, :, pl.ds(c * chunk, chunk)].astype(jnp.bfloat16)
                acc = acc + lax.dot_general(
                    xw[j], W_j, (((1,), (0,)), ((), ())),
                    preferred_element_type=jnp.float32,
                )
            return acc

        send_buf[0, :, :] = pchunk(my)
        acc = None
        for s in range(N_DEV - 1):
            rdma = pltpu.make_async_remote_copy(
                src_ref=send_buf.at[s],
                dst_ref=recv_buf.at[s],
                send_sem=rs_send.at[s],
                recv_sem=rs_recv.at[s],
                device_id=(right,),
                device_id_type=pl.DeviceIdType.MESH,
            )
            rdma.start()
            nxt = pchunk(lax.rem(my + N_DEV - 1 - s, N_DEV))
            rdma.wait()
            acc = recv_buf[s, :, :] + nxt
            if s < N_DEV - 2:
                send_buf[s + 1, :, :] = acc

        own_c = lax.rem(my + 1, N_DEV)
        ag_buf[0, :, :] = acc
        out_ref[:, pl.ds(own_c * chunk, chunk)] = acc

        for t in range(N_DEV - 1):
            rdma = pltpu.make_async_remote_copy(
                src_ref=ag_buf.at[t],
                dst_ref=ag_buf.at[t + 1],
                send_sem=ag_send.at[t],
                recv_sem=ag_recv.at[t],
                device_id=(right,),
                device_id_type=pl.DeviceIdType.MESH,
            )
            rdma.start()
            rdma.wait()
            c = lax.rem(my + N_DEV - t, N_DEV)
            out_ref[:, pl.ds(c * chunk, chunk)] = ag_buf[t + 1, :, :]

    return pl.pallas_call(
        body,
        out_shape=jax.ShapeDtypeStruct((n_tok, d_out), jnp.float32),
        in_specs=[
            pl.BlockSpec(memory_space=pltpu.VMEM),
            pl.BlockSpec(memory_space=pltpu.VMEM),
            pl.BlockSpec(memory_space=pltpu.VMEM),
            pl.BlockSpec(memory_space=pltpu.VMEM),
        ],
        out_specs=pl.BlockSpec(memory_space=pltpu.VMEM),
        scratch_shapes=[
            pltpu.VMEM((N_DEV - 1, n_tok, chunk), jnp.float32),
            pltpu.VMEM((N_DEV - 1, n_tok, chunk), jnp.float32),
            pltpu.VMEM((N_DEV, n_tok, chunk), jnp.float32),
            pltpu.SemaphoreType.DMA((N_DEV - 1,)),
            pltpu.SemaphoreType.DMA((N_DEV - 1,)),
            pltpu.SemaphoreType.DMA((N_DEV - 1,)),
            pltpu.SemaphoreType.DMA((N_DEV - 1,)),
        ],
        compiler_params=pltpu.CompilerParams(collective_id=0),
    )(x, router_W, route_idx, expert_W)


# device time: 60419 ns/iter; 2.6955x vs baseline; 1.6038x over previous
import jax
import jax.numpy as jnp
from jax import lax
from jax.experimental import pallas as pl
from jax.experimental.pallas import tpu as pltpu

N_DEV = 4
N_LOCAL_EXPERTS = 4


def kernel(x, router_W, route_idx, expert_W):
    n_tok, d_model = x.shape
    n_experts = router_W.shape[1]
    d_out = expert_W.shape[2]
    chunk = d_out // N_DEV
    half = n_tok // 2

    def body(x_ref, rw_ref, idx_ref, ew_ref, out_ref,
             a_send, a_recv, a_ag, b_send, b_recv, b_ag,
             a_rs_ss, a_rs_rs, a_ag_ss, a_ag_rs,
             b_rs_ss, b_rs_rs, b_ag_ss, b_ag_rs):
        my = lax.axis_index("i")
        left = lax.rem(my + N_DEV - 1, N_DEV)
        right = lax.rem(my + 1, N_DEV)

        barrier_sem = pltpu.get_barrier_semaphore()
        for nbr in (left, right):
            pl.semaphore_signal(
                barrier_sem, inc=1,
                device_id=(nbr,), device_id_type=pl.DeviceIdType.MESH,
            )
        pl.semaphore_wait(barrier_sem, 2)

        xf = x_ref[:, :]
        scores = lax.dot_general(
            xf, rw_ref[:, :], (((1,), (0,)), ((), ())),
            preferred_element_type=jnp.float32,
        )
        s_max = jnp.max(scores, axis=1, keepdims=True)
        probs = jnp.exp(scores - s_max)
        probs = probs / jnp.sum(probs, axis=1, keepdims=True)

        e0 = idx_ref[:, 0:1]
        e1 = idx_ref[:, 1:2]
        expert_iota = lax.broadcasted_iota(jnp.int32, (n_tok, n_experts), 1)
        g0 = jnp.sum(
            jnp.where(e0 == expert_iota, probs, 0.0), axis=1, keepdims=True
        )
        g1 = jnp.sum(
            jnp.where(e1 == expert_iota, probs, 0.0), axis=1, keepdims=True
        )
        gs = g0 + g1
        g0n = g0 / gs
        g1n = g1 / gs

        xw = []
        for j in range(N_LOCAL_EXPERTS):
            gid = my * N_LOCAL_EXPERTS + j
            w_j = (
                g0n * (e0 == gid).astype(jnp.float32)
                + g1n * (e1 == gid).astype(jnp.float32)
            )
            xw.append((xf * w_j).astype(jnp.bfloat16))

        def pchunk(c, r0):
            acc = jnp.zeros((half, chunk), dtype=jnp.float32)
            for j in range(N_LOCAL_EXPERTS):
                W_j = ew_ref[j, :, pl.ds(c * chunk, chunk)].astype(jnp.bfloat16)
                acc = acc + lax.dot_general(
                    xw[j][r0:r0 + half], W_j, (((1,), (0,)), ((), ())),
                    preferred_element_type=jnp.float32,
                )
            return acc

        def copy(src, dst, ss, rs, dev):
            return pltpu.make_async_remote_copy(
                src_ref=src, dst_ref=dst, send_sem=ss, recv_sem=rs,
                device_id=(dev,), device_id_type=pl.DeviceIdType.MESH,
            )

        a_send[0, :, :] = pchunk(my, 0)
        b_send[0, :, :] = pchunk(my, half)
        acc_a = acc_b = None
        for s in range(N_DEV - 1):
            rda = copy(a_send.at[s], a_recv.at[s],
                       a_rs_ss.at[s], a_rs_rs.at[s], right)
            rdb = copy(b_send.at[s], b_recv.at[s],
                       b_rs_ss.at[s], b_rs_rs.at[s], left)
            rda.start()
            rdb.start()
            nxt_a = pchunk(lax.rem(my + N_DEV - 1 - s, N_DEV), 0)
            nxt_b = pchunk(lax.rem(my + 1 + s, N_DEV), half)
            rda.wait()
            acc_a = a_recv[s, :, :] + nxt_a
            rdb.wait()
            acc_b = b_recv[s, :, :] + nxt_b
            if s < N_DEV - 2:
                a_send[s + 1, :, :] = acc_a
                b_send[s + 1, :, :] = acc_b

        c_a = lax.rem(my + 1, N_DEV)
        c_b = lax.rem(my + N_DEV - 1, N_DEV)
        a_ag[0, :, :] = acc_a
        b_ag[0, :, :] = acc_b
        out_ref[0:half, pl.ds(c_a * chunk, chunk)] = acc_a
        out_ref[half:n_tok, pl.ds(c_b * chunk, chunk)] = acc_b

        for t in range(N_DEV - 1):
            rda = copy(a_ag.at[t], a_ag.at[t + 1],
                       a_ag_ss.at[t], a_ag_rs.at[t], right)
            rdb = copy(b_ag.at[t], b_ag.at[t + 1],
                       b_ag_ss.at[t], b_ag_rs.at[t], left)
            rda.start()
            rdb.start()
            rda.wait()
            ca = lax.rem(my + N_DEV - t, N_DEV)
            out_ref[0:half, pl.ds(ca * chunk, chunk)] = a_ag[t + 1, :, :]
            rdb.wait()
            cb = lax.rem(my + t, N_DEV)
            out_ref[half:n_tok, pl.ds(cb * chunk, chunk)] = b_ag[t + 1, :, :]

    half_buf = lambda n: pltpu.VMEM((n, half, chunk), jnp.float32)
    sems = lambda: pltpu.SemaphoreType.DMA((N_DEV - 1,))
    return pl.pallas_call(
        body,
        out_shape=jax.ShapeDtypeStruct((n_tok, d_out), jnp.float32),
        in_specs=[
            pl.BlockSpec(memory_space=pltpu.VMEM),
            pl.BlockSpec(memory_space=pltpu.VMEM),
            pl.BlockSpec(memory_space=pltpu.VMEM),
            pl.BlockSpec(memory_space=pltpu.VMEM),
        ],
        out_specs=pl.BlockSpec(memory_space=pltpu.VMEM),
        scratch_shapes=[
            half_buf(N_DEV - 1),
            half_buf(N_DEV - 1),
            half_buf(N_DEV),
            half_buf(N_DEV - 1),
            half_buf(N_DEV - 1),
            half_buf(N_DEV),
            sems(), sems(), sems(), sems(),
            sems(), sems(), sems(), sems(),
        ],
        compiler_params=pltpu.CompilerParams(collective_id=0),
    )(x, router_W, route_idx, expert_W)


# device time: 43890 ns/iter; 3.7106x vs baseline; 1.3766x over previous
import jax
import jax.numpy as jnp
from jax import lax
from jax.experimental import pallas as pl
from jax.experimental.pallas import tpu as pltpu

N_DEV = 4
N_LOCAL_EXPERTS = 4


def kernel(x, router_W, route_idx, expert_W):
    n_tok, d_model = x.shape
    n_experts = router_W.shape[1]
    d_out = expert_W.shape[2]
    chunk = d_out // N_DEV
    half = n_tok // 2

    def body(x_ref, rw_ref, idx_ref, ew_ref, out_ref,
             a_send, a_recv, a_ag, b_send, b_recv, b_ag,
             a_rs_ss, a_rs_rs, a_ag_ss, a_ag_rs,
             b_rs_ss, b_rs_rs, b_ag_ss, b_ag_rs):
        my = lax.axis_index("i")
        left = lax.rem(my + N_DEV - 1, N_DEV)
        right = lax.rem(my + 1, N_DEV)

        barrier_sem = pltpu.get_barrier_semaphore()
        for nbr in (left, right):
            pl.semaphore_signal(
                barrier_sem, inc=1,
                device_id=(nbr,), device_id_type=pl.DeviceIdType.MESH,
            )
        pl.semaphore_wait(barrier_sem, 2)

        xf = x_ref[:, :]
        scores = lax.dot_general(
            xf, rw_ref[:, :], (((1,), (0,)), ((), ())),
            preferred_element_type=jnp.float32,
        )
        s_max = jnp.max(scores, axis=1, keepdims=True)
        probs = jnp.exp(scores - s_max)
        probs = probs / jnp.sum(probs, axis=1, keepdims=True)

        e0 = idx_ref[:, 0:1]
        e1 = idx_ref[:, 1:2]
        expert_iota = lax.broadcasted_iota(jnp.int32, (n_tok, n_experts), 1)
        g0 = jnp.sum(
            jnp.where(e0 == expert_iota, probs, 0.0), axis=1, keepdims=True
        )
        g1 = jnp.sum(
            jnp.where(e1 == expert_iota, probs, 0.0), axis=1, keepdims=True
        )
        gs = g0 + g1
        g0n = g0 / gs
        g1n = g1 / gs

        xw = []
        for j in range(N_LOCAL_EXPERTS):
            gid = my * N_LOCAL_EXPERTS + j
            w_j = (
                g0n * (e0 == gid).astype(jnp.float32)
                + g1n * (e1 == gid).astype(jnp.float32)
            )
            xw.append((xf * w_j).astype(jnp.bfloat16))

        def pchunk(c, r0):
            acc = jnp.zeros((half, chunk), dtype=jnp.float32)
            for j in range(N_LOCAL_EXPERTS):
                W_j = ew_ref[j, :, pl.ds(c * chunk, chunk)].astype(jnp.bfloat16)
                acc = acc + lax.dot_general(
                    xw[j][r0:r0 + half], W_j, (((1,), (0,)), ((), ())),
                    preferred_element_type=jnp.float32,
                )
            return acc

        def copy(src, dst, ss, rs, dev):
            return pltpu.make_async_remote_copy(
                src_ref=src, dst_ref=dst, send_sem=ss, recv_sem=rs,
                device_id=(dev,), device_id_type=pl.DeviceIdType.MESH,
            )

        a_send[0, :, :] = pchunk(my, 0).astype(jnp.bfloat16)
        b_send[0, :, :] = pchunk(my, half).astype(jnp.bfloat16)
        acc_a = acc_b = None
        for s in range(N_DEV - 1):
            rda = copy(a_send.at[s], a_recv.at[s],
                       a_rs_ss.at[s], a_rs_rs.at[s], right)
            rdb = copy(b_send.at[s], b_recv.at[s],
                       b_rs_ss.at[s], b_rs_rs.at[s], left)
            rda.start()
            rdb.start()
            nxt_a = pchunk(lax.rem(my + N_DEV - 1 - s, N_DEV), 0)
            nxt_b = pchunk(lax.rem(my + 1 + s, N_DEV), half)
            rda.wait()
            acc_a = a_recv[s, :, :].astype(jnp.float32) + nxt_a
            rdb.wait()
            acc_b = b_recv[s, :, :].astype(jnp.float32) + nxt_b
            if s < N_DEV - 2:
                a_send[s + 1, :, :] = acc_a.astype(jnp.bfloat16)
                b_send[s + 1, :, :] = acc_b.astype(jnp.bfloat16)

        c_a = lax.rem(my + 1, N_DEV)
        c_b = lax.rem(my + N_DEV - 1, N_DEV)
        a_ag[0, :, :] = acc_a.astype(jnp.bfloat16)
        b_ag[0, :, :] = acc_b.astype(jnp.bfloat16)
        out_ref[0:half, pl.ds(c_a * chunk, chunk)] = acc_a
        out_ref[half:n_tok, pl.ds(c_b * chunk, chunk)] = acc_b

        for t in range(N_DEV - 1):
            rda = copy(a_ag.at[t], a_ag.at[t + 1],
                       a_ag_ss.at[t], a_ag_rs.at[t], right)
            rdb = copy(b_ag.at[t], b_ag.at[t + 1],
                       b_ag_ss.at[t], b_ag_rs.at[t], left)
            rda.start()
            rdb.start()
            rda.wait()
            ca = lax.rem(my + N_DEV - t, N_DEV)
            out_ref[0:half, pl.ds(ca * chunk, chunk)] = (
                a_ag[t + 1, :, :].astype(jnp.float32)
            )
            rdb.wait()
            cb = lax.rem(my + t, N_DEV)
            out_ref[half:n_tok, pl.ds(cb * chunk, chunk)] = (
                b_ag[t + 1, :, :].astype(jnp.float32)
            )

    half_buf = lambda n: pltpu.VMEM((n, half, chunk), jnp.bfloat16)
    sems = lambda: pltpu.SemaphoreType.DMA((N_DEV - 1,))
    return pl.pallas_call(
        body,
        out_shape=jax.ShapeDtypeStruct((n_tok, d_out), jnp.float32),
        in_specs=[
            pl.BlockSpec(memory_space=pltpu.VMEM),
            pl.BlockSpec(memory_space=pltpu.VMEM),
            pl.BlockSpec(memory_space=pltpu.VMEM),
            pl.BlockSpec(memory_space=pltpu.VMEM),
        ],
        out_specs=pl.BlockSpec(memory_space=pltpu.VMEM),
        scratch_shapes=[
            half_buf(N_DEV - 1),
            half_buf(N_DEV - 1),
            half_buf(N_DEV),
            half_buf(N_DEV - 1),
            half_buf(N_DEV - 1),
            half_buf(N_DEV),
            sems(), sems(), sems(), sems(),
            sems(), sems(), sems(), sems(),
        ],
        compiler_params=pltpu.CompilerParams(collective_id=0),
    )(x, router_W, route_idx, expert_W)
